# baseline (speedup 1.0000x reference)
import functools

import jax
import jax.numpy as jnp
from jax import lax
from jax.experimental import pallas as pl
from jax.experimental.pallas import tpu as pltpu
from jax.experimental.pallas import tpu_sc as plsc

ROWS = 128
COLS = 32768
NUM_CORES = 2
NUM_SUBCORES = 16
LANES = 16
NUM_WORKERS = NUM_CORES * NUM_SUBCORES
ROWS_PER_WORKER = ROWS // NUM_WORKERS
SEG = COLS // LANES
UNROLL = 8
NBUF = 3


def _scan_row(buf):
  idx0 = lax.iota(jnp.int32, LANES) * SEG
  zero = jnp.zeros((LANES,), jnp.float32)

  def body_a(t, accs):
    k = t * UNROLL
    out = []
    for u, a in enumerate(accs):
      v = plsc.load_gather(buf, [idx0 + (k + u)])
      out.append(a + v)
    return tuple(out)

  accs = lax.fori_loop(0, SEG // UNROLL, body_a, (zero,) * UNROLL)
  tot = accs[0]
  for a in accs[1:]:
    tot = tot + a
  run0 = plsc.cumsum(tot) - tot

  def body_b(t, run):
    k = t * UNROLL
    for u in range(UNROLL):
      idx = idx0 + (k + u)
      v = plsc.load_gather(buf, [idx])
      run = run + v
      plsc.store_scatter(buf, [idx], run)
    return run

  lax.fori_loop(0, SEG // UNROLL, body_b, run0)


def _body(x_hbm, out_hbm, b0, b1, b2, si0, si1, si2, so0, so1, so2):
  bufs = (b0, b1, b2)
  sin = (si0, si1, si2)
  sout = (so0, so1, so2)
  wid = lax.axis_index("s") * NUM_CORES + lax.axis_index("c")
  base = wid * ROWS_PER_WORKER

  ins = [
      pltpu.async_copy(x_hbm.at[base + i], bufs[i], sin[i])
      for i in range(min(NBUF, ROWS_PER_WORKER))
  ]
  outs = [None] * ROWS_PER_WORKER
  out_waited = [False] * ROWS_PER_WORKER
  for i in range(ROWS_PER_WORKER):
    nxt = i - 1 + NBUF
    if i >= 1 and nxt < ROWS_PER_WORKER:
      outs[nxt - NBUF].wait()
      out_waited[nxt - NBUF] = True
      ins.append(
          pltpu.async_copy(x_hbm.at[base + nxt], bufs[nxt % NBUF],
                           sin[nxt % NBUF]))
    ins[i].wait()
    _scan_row(bufs[i % NBUF])
    outs[i] = pltpu.async_copy(bufs[i % NBUF], out_hbm.at[base + i],
                               sout[i % NBUF])
  for i in range(ROWS_PER_WORKER):
    if not out_waited[i]:
      outs[i].wait()


_cumsum_sc = functools.partial(
    pl.kernel,
    out_type=jax.ShapeDtypeStruct((ROWS, COLS), jnp.float32),
    mesh=plsc.VectorSubcoreMesh(core_axis_name="c", subcore_axis_name="s"),
    scratch_types=[
        pltpu.VMEM((COLS,), jnp.float32),
        pltpu.VMEM((COLS,), jnp.float32),
        pltpu.VMEM((COLS,), jnp.float32),
        pltpu.SemaphoreType.DMA,
        pltpu.SemaphoreType.DMA,
        pltpu.SemaphoreType.DMA,
        pltpu.SemaphoreType.DMA,
        pltpu.SemaphoreType.DMA,
        pltpu.SemaphoreType.DMA,
    ],
    compiler_params=pltpu.CompilerParams(needs_layout_passes=False),
)(_body)


def kernel(x):
  return _cumsum_sc(x)
